# baseline (speedup 1.0000x reference)
import functools

import jax
import jax.numpy as jnp
from jax import lax
from jax.experimental import pallas as pl
from jax.experimental.pallas import tpu as pltpu
from jax.experimental.pallas import tpu_sc as plsc

_NC = 2
_NS = 16


def _sc_copy(table_hbm, out_hbm, buf, in_sems, out_sems, *, rows_per_c, chunk,
             nslots):
    cid = lax.axis_index("c")
    sid = lax.axis_index("s")
    base = cid * rows_per_c
    n = rows_per_c // chunk

    def in_copy(j, slot):
        return pltpu.make_async_copy(
            table_hbm.at[pl.ds(base + j * chunk, chunk), :], buf.at[slot],
            in_sems.at[slot])

    def out_copy(j, slot):
        return pltpu.make_async_copy(
            buf.at[slot], out_hbm.at[pl.ds(base + j * chunk, chunk), :],
            out_sems.at[slot])

    @pl.when(sid == 0)
    def _():
        for j in range(min(nslots, n)):
            in_copy(j, j).start()
        for j in range(n):
            if j >= 1 and j + nslots - 1 < n:
                out_copy(j - 1, (j - 1) % nslots).wait()
                in_copy(j + nslots - 1, (j - 1) % nslots).start()
            in_copy(j, j % nslots).wait()
            out_copy(j, j % nslots).start()
        for j in range(max(0, n - nslots), n):
            out_copy(j, j % nslots).wait()


def kernel(inputs, table):
    seq_len = inputs.shape[-1]
    rows, dim = table.shape
    assert seq_len == rows
    rows_per_c = rows // _NC
    chunk = 512
    nslots = 3
    mesh = plsc.VectorSubcoreMesh(core_axis_name="c", subcore_axis_name="s")
    f = functools.partial(_sc_copy, rows_per_c=rows_per_c, chunk=chunk,
                          nslots=nslots)
    return pl.kernel(
        f,
        mesh=mesh,
        out_type=jax.ShapeDtypeStruct((rows, dim), table.dtype),
        scratch_types=[
            pltpu.MemorySpace.VMEM_SHARED((nslots, chunk, dim), jnp.float32),
            pltpu.SemaphoreType.DMA((nslots,)),
            pltpu.SemaphoreType.DMA((nslots,)),
        ],
    )(table)


# speedup vs baseline: 3.0113x; 3.0113x over previous
import jax
import jax.numpy as jnp
from jax.experimental import pallas as pl


def _copy_block(table_ref, out_ref):
    out_ref[...] = table_ref[...]


def kernel(inputs, table):
    seq_len = inputs.shape[-1]
    rows, dim = table.shape
    assert seq_len == rows
    block_rows = 2048
    grid = (rows // block_rows,)
    return pl.pallas_call(
        _copy_block,
        grid=grid,
        in_specs=[pl.BlockSpec((block_rows, dim), lambda i: (i, 0))],
        out_specs=pl.BlockSpec((block_rows, dim), lambda i: (i, 0)),
        out_shape=jax.ShapeDtypeStruct((rows, dim), table.dtype),
    )(table)
